# baseline (speedup 1.0000x reference)
import functools

import jax
import jax.numpy as jnp
from jax import lax
from jax.experimental import pallas as pl
from jax.experimental.pallas import tpu as pltpu
from jax.experimental.pallas import tpu_sc as plsc

N = 38333
NP = 38400
E = 1226656
C = 1024
CR = C // 128
NCH0 = 38
NCH1 = 37
NCHT = NCH0 + NCH1
EP = 16 * NCHT * C
RPS = NP // 16
BT = 2400
GRID = NP // BT


def _mesh():
  return plsc.VectorSubcoreMesh(core_axis_name="c", subcore_axis_name="s")


def _sc_prep(emb, ids0, ids1, dstr, zeros1, ones1):
  ed = emb.shape[1]
  idrows = NP // 128

  def body(emb_h, ids0_h, ids1_h, dst_h, z_h, one_h, ef0_h, ef1_h, deg_h,
           idx_v, er0_v, er1_v, dst_v, ones_v, dacc, sems, ssems, isems):
    cid = lax.axis_index("c")
    sid = lax.axis_index("s")
    pltpu.sync_copy(z_h.at[pl.ds(sid * RPS, RPS)],
                    dacc.at[pl.ds(sid * RPS, RPS)])
    pltpu.sync_copy(one_h, ones_v)

    nr = lax.select(cid == 0, 14, 5)
    rbase = cid * (16 * 14) + sid * nr

    def erow(t, carry):
      row = rbase + t

      @pl.when(row < idrows)
      def _():
        @pl.when(t > 0)
        def _():
          rowp = row - 1
          pltpu.make_async_copy(er0_v, ef0_h.at[pl.ds(rowp * 128, 128)],
                                sems.at[2]).wait()
          pltpu.make_async_copy(er1_v, ef1_h.at[pl.ds(rowp * 128, 128)],
                                sems.at[3]).wait()
        pltpu.sync_copy(ids0_h.at[pl.ds(row, 1)], idx_v.at[pl.ds(0, 1)])
        pltpu.sync_copy(ids1_h.at[pl.ds(row, 1)], idx_v.at[pl.ds(1, 1)])
        g0 = pltpu.async_copy(emb_h.at[idx_v.at[0]], er0_v, sems.at[0])
        g1 = pltpu.async_copy(emb_h.at[idx_v.at[1]], er1_v, sems.at[1])
        g0.wait()
        pltpu.async_copy(er0_v, ef0_h.at[pl.ds(row * 128, 128)], sems.at[2])
        g1.wait()
        pltpu.async_copy(er1_v, ef1_h.at[pl.ds(row * 128, 128)], sems.at[3])
      return carry

    lax.fori_loop(0, nr, erow, 0)

    @pl.when(rbase < idrows)
    def _():
      rowl = lax.min(rbase + nr, idrows) - 1
      pltpu.make_async_copy(er0_v, ef0_h.at[pl.ds(rowl * 128, 128)],
                            sems.at[2]).wait()
      pltpu.make_async_copy(er1_v, ef1_h.at[pl.ds(rowl * 128, 128)],
                            sems.at[3]).wait()
    plsc.subcore_barrier()

    nc = lax.select(cid == 0, NCH0, NCH1)
    ebase = cid * (16 * NCH0 * CR) + sid * nc * CR
    pltpu.async_copy(dst_h.at[pl.ds(ebase, CR)], dst_v.at[0], isems.at[0])

    def chunk(k, carry):
      p = lax.rem(k, 2)
      pn = lax.rem(k + 1, 2)
      row0 = ebase + k * CR
      pltpu.make_async_copy(dst_h.at[pl.ds(row0, CR)], dst_v.at[p],
                            isems.at[p]).wait()
      for j in range(CR):
        @pl.when(k > 0)
        def _(j=j, p=p):
          pltpu.make_async_copy(ones_v, dacc.at[dst_v.at[p, j]],
                                ssems.at[j]).wait()
        pltpu.async_copy(ones_v, dacc.at[dst_v.at[p, j]], ssems.at[j],
                         add=True)

      @pl.when(k + 1 < nc)
      def _():
        rown = ebase + (k + 1) * CR
        pltpu.async_copy(dst_h.at[pl.ds(rown, CR)], dst_v.at[pn],
                         isems.at[pn])
      return carry

    lax.fori_loop(0, nc, chunk, 0)
    pl_ = lax.rem(nc - 1, 2)
    for j in range(CR):
      pltpu.make_async_copy(ones_v, dacc.at[dst_v.at[pl_, j]],
                            ssems.at[j]).wait()
    plsc.subcore_barrier()
    pltpu.sync_copy(dacc.at[pl.ds(sid * RPS, RPS)],
                    deg_h.at[cid, pl.ds(sid * RPS, RPS)])

  k = pl.kernel(
      body,
      out_type=(jax.ShapeDtypeStruct((NP, ed), jnp.float32),
                jax.ShapeDtypeStruct((NP, ed), jnp.float32),
                jax.ShapeDtypeStruct((2, NP, 16), jnp.float32)),
      mesh=_mesh(),
      compiler_params=pltpu.CompilerParams(use_tc_tiling_on_sc=False),
      scratch_types=[pltpu.VMEM((2, 128), jnp.int32),
                     pltpu.VMEM((128, ed), jnp.float32),
                     pltpu.VMEM((128, ed), jnp.float32),
                     pltpu.VMEM((2, CR, 128), jnp.int32),
                     pltpu.VMEM((128, 16), jnp.float32),
                     pltpu.VMEM_SHARED((NP, 16), jnp.float32),
                     pltpu.SemaphoreType.DMA((4,)),
                     pltpu.SemaphoreType.DMA((CR,)),
                     pltpu.SemaphoreType.DMA((2,))])
  return k(emb, ids0, ids1, dstr, zeros1, ones1)


def _sc_spmm(g, srcr, dstr, zeros):
  f_dim = g.shape[1]

  def body(g_h, src_h, dst_h, z_h, out_h, src_v, dst_v, rows_v, acc,
           gsems, ssems, isems):
    cid = lax.axis_index("c")
    sid = lax.axis_index("s")
    nc = lax.select(cid == 0, NCH0, NCH1)
    base = cid * (16 * NCH0 * CR) + sid * nc * CR

    pltpu.async_copy(src_h.at[pl.ds(base, CR)], src_v.at[0], isems.at[0])
    pltpu.async_copy(dst_h.at[pl.ds(base, CR)], dst_v.at[0], isems.at[1])
    pltpu.sync_copy(z_h.at[pl.ds(sid * RPS, RPS)],
                    acc.at[pl.ds(sid * RPS, RPS)])
    plsc.subcore_barrier()

    def chunk(k, carry):
      p = lax.rem(k, 2)
      pn = lax.rem(k + 1, 2)
      row0 = base + k * CR
      pltpu.make_async_copy(src_h.at[pl.ds(row0, CR)], src_v.at[p],
                            isems.at[2 * p]).wait()
      pltpu.make_async_copy(dst_h.at[pl.ds(row0, CR)], dst_v.at[p],
                            isems.at[2 * p + 1]).wait()
      gd = []
      for j in range(CR):
        @pl.when(k > 0)
        def _(j=j, p=p):
          pltpu.make_async_copy(rows_v.at[pl.ds(j * 128, 128)],
                                acc.at[dst_v.at[p, j]], ssems.at[j]).wait()
        gd.append(pltpu.async_copy(g_h.at[src_v.at[p, j]],
                                   rows_v.at[pl.ds(j * 128, 128)],
                                   gsems.at[j]))
      @pl.when(k + 1 < nc)
      def _():
        rown = base + (k + 1) * CR
        pltpu.async_copy(src_h.at[pl.ds(rown, CR)], src_v.at[pn],
                         isems.at[2 * pn])
        pltpu.async_copy(dst_h.at[pl.ds(rown, CR)], dst_v.at[pn],
                         isems.at[2 * pn + 1])
      for j in range(CR):
        gd[j].wait()
        pltpu.async_copy(rows_v.at[pl.ds(j * 128, 128)],
                         acc.at[dst_v.at[p, j]], ssems.at[j], add=True)
      return carry

    lax.fori_loop(0, nc, chunk, 0)
    pl_ = lax.rem(nc - 1, 2)
    for j in range(CR):
      pltpu.make_async_copy(rows_v.at[pl.ds(j * 128, 128)],
                            acc.at[dst_v.at[pl_, j]], ssems.at[j]).wait()
    plsc.subcore_barrier()
    pltpu.sync_copy(acc.at[pl.ds(sid * RPS, RPS)],
                    out_h.at[cid, pl.ds(sid * RPS, RPS)])

  k = pl.kernel(
      body,
      out_type=jax.ShapeDtypeStruct((2, NP, f_dim), jnp.float32),
      mesh=_mesh(),
      compiler_params=pltpu.CompilerParams(use_tc_tiling_on_sc=False),
      scratch_types=[pltpu.VMEM((2, CR, 128), jnp.int32),
                     pltpu.VMEM((2, CR, 128), jnp.int32),
                     pltpu.VMEM((C, f_dim), jnp.float32),
                     pltpu.VMEM_SHARED((NP, f_dim), jnp.float32),
                     pltpu.SemaphoreType.DMA((CR,)),
                     pltpu.SemaphoreType.DMA((CR,)),
                     pltpu.SemaphoreType.DMA((4,))])
  return k(g, srcr, dstr, zeros)


def _lrelu(v):
  return jnp.where(v > 0, v, 0.01 * v)


def _rowmask(i):
  rows = i * BT + lax.broadcasted_iota(jnp.int32, (BT, 1), 0)
  return rows < N


def _tc1(ef0, ef1, xf, degp, w1a, w1b, w1c):
  def body(ef0_r, ef1_r, xf_r, deg_r, wa_r, wb_r, wc_r, g1a_r, g1b_r, dinv_r):
    i = pl.program_id(0)
    deg = deg_r[0][:, 0:1] + deg_r[1][:, 0:1] + 1.0
    dinv = 1.0 / jnp.sqrt(deg)
    h = (jnp.dot(ef0_r[...], wa_r[...], preferred_element_type=jnp.float32)
         + jnp.dot(ef1_r[...], wb_r[...], preferred_element_type=jnp.float32)
         + jnp.dot(xf_r[...], wc_r[...], preferred_element_type=jnp.float32))
    g = jnp.where(_rowmask(i), dinv * h, 0.0)
    g1a_r[...] = g[:, :32]
    g1b_r[...] = g[:, 32:]
    dinv_r[...] = dinv

  ed = ef0.shape[1]
  return pl.pallas_call(
      body,
      grid=(GRID,),
      in_specs=[
          pl.BlockSpec((BT, ed), lambda i: (i, 0)),
          pl.BlockSpec((BT, ed), lambda i: (i, 0)),
          pl.BlockSpec((BT, 3), lambda i: (i, 0)),
          pl.BlockSpec((2, BT, 16), lambda i: (0, i, 0)),
          pl.BlockSpec((ed, 64), lambda i: (0, 0)),
          pl.BlockSpec((ed, 64), lambda i: (0, 0)),
          pl.BlockSpec((3, 64), lambda i: (0, 0)),
      ],
      out_specs=[
          pl.BlockSpec((BT, 32), lambda i: (i, 0)),
          pl.BlockSpec((BT, 32), lambda i: (i, 0)),
          pl.BlockSpec((BT, 1), lambda i: (i, 0)),
      ],
      out_shape=[
          jax.ShapeDtypeStruct((NP, 32), jnp.float32),
          jax.ShapeDtypeStruct((NP, 32), jnp.float32),
          jax.ShapeDtypeStruct((NP, 1), jnp.float32),
      ],
  )(ef0, ef1, xf, degp, w1a, w1b, w1c)


def _tc2(s1a, s1b, g1a, g1b, dinv, b1, w2):
  def body(sa_r, sb_r, ga_r, gb_r, dinv_r, b_r, w_r, out_r):
    i = pl.program_id(0)
    s = jnp.concatenate([sa_r[0] + sa_r[1] + ga_r[...],
                         sb_r[0] + sb_r[1] + gb_r[...]], axis=1)
    t = dinv_r[...] * s + b_r[...]
    f = _lrelu(t)
    out_r[...] = jnp.where(
        _rowmask(i),
        dinv_r[...] * jnp.dot(f, w_r[...], preferred_element_type=jnp.float32),
        0.0)

  return pl.pallas_call(
      body,
      grid=(GRID,),
      in_specs=[
          pl.BlockSpec((2, BT, 32), lambda i: (0, i, 0)),
          pl.BlockSpec((2, BT, 32), lambda i: (0, i, 0)),
          pl.BlockSpec((BT, 32), lambda i: (i, 0)),
          pl.BlockSpec((BT, 32), lambda i: (i, 0)),
          pl.BlockSpec((BT, 1), lambda i: (i, 0)),
          pl.BlockSpec((1, 64), lambda i: (0, 0)),
          pl.BlockSpec((64, 32), lambda i: (0, 0)),
      ],
      out_specs=pl.BlockSpec((BT, 32), lambda i: (i, 0)),
      out_shape=jax.ShapeDtypeStruct((NP, 32), jnp.float32),
  )(s1a, s1b, g1a, g1b, dinv, b1, w2)


def _tc_mid(s, g, dinv, b, w, residual, f_out):
  f_in = g.shape[1]

  def body(s_r, g_r, dinv_r, b_r, w_r, out_r):
    i = pl.program_id(0)
    t = dinv_r[...] * (s_r[0] + s_r[1] + g_r[...]) + b_r[...]
    f = _lrelu(t) + t if residual else _lrelu(t)
    out_r[...] = jnp.where(
        _rowmask(i),
        dinv_r[...] * jnp.dot(f, w_r[...], preferred_element_type=jnp.float32),
        0.0)

  return pl.pallas_call(
      body,
      grid=(GRID,),
      in_specs=[
          pl.BlockSpec((2, BT, f_in), lambda i: (0, i, 0)),
          pl.BlockSpec((BT, f_in), lambda i: (i, 0)),
          pl.BlockSpec((BT, 1), lambda i: (i, 0)),
          pl.BlockSpec((1, f_in), lambda i: (0, 0)),
          pl.BlockSpec((f_in, f_out), lambda i: (0, 0)),
      ],
      out_specs=pl.BlockSpec((BT, f_out), lambda i: (i, 0)),
      out_shape=jax.ShapeDtypeStruct((NP, f_out), jnp.float32),
  )(s, g, dinv, b, w)


def _tc_final(s5, g5, dinv, b5, fcw1, fcb1, fcw2, fcb2):
  def body(s_r, g_r, dinv_r, b5_r, w1_r, fb1_r, w2_r, fb2_r, out_r, acc):
    i = pl.program_id(0)

    @pl.when(i == 0)
    def _():
      acc[...] = jnp.zeros_like(acc)

    mask = _rowmask(i)
    t = (dinv_r[...] * (s_r[0][:, 0:1] + s_r[1][:, 0:1] + g_r[:, 0:1])
         + b5_r[...])
    f = jnp.where(mask, _lrelu(t), 0.0)
    w = jnp.where(mask, w1_r[...], 0.0)
    acc[...] += lax.dot_general(f, w, (((0,), (0,)), ((), ())),
                                preferred_element_type=jnp.float32)

    @pl.when(i == GRID - 1)
    def _():
      h = jnp.maximum(acc[...] + fb1_r[...], 0.0)
      out_r[...] = jnp.maximum(
          jnp.dot(h, w2_r[...], preferred_element_type=jnp.float32)
          + fb2_r[...], 0.0)

  return pl.pallas_call(
      body,
      grid=(GRID,),
      in_specs=[
          pl.BlockSpec((2, BT, 16), lambda i: (0, i, 0)),
          pl.BlockSpec((BT, 16), lambda i: (i, 0)),
          pl.BlockSpec((BT, 1), lambda i: (i, 0)),
          pl.BlockSpec((1, 1), lambda i: (0, 0)),
          pl.BlockSpec((BT, 128), lambda i: (i, 0)),
          pl.BlockSpec((1, 128), lambda i: (0, 0)),
          pl.BlockSpec((128, 128), lambda i: (0, 0)),
          pl.BlockSpec((1, 128), lambda i: (0, 0)),
      ],
      out_specs=pl.BlockSpec((1, 128), lambda i: (0, 0)),
      out_shape=jax.ShapeDtypeStruct((1, 128), jnp.float32),
      scratch_shapes=[pltpu.VMEM((1, 128), jnp.float32)],
  )(s5, g5, dinv, b5, fcw1, fcb1, fcw2, fcb2)


def kernel(x, edge_index, emb, W1, b1, W2, b2, W3, b3, W4, b4, W5, b5,
           fcW1, fcb1, fcW2, fcb2):
  src = edge_index[0].astype(jnp.int32)
  dst = edge_index[1].astype(jnp.int32)
  pad_e = jnp.full((EP - E,), N, jnp.int32)
  srcr = jnp.concatenate([src, pad_e]).reshape(EP // 128, 128)
  dstr = jnp.concatenate([dst, pad_e]).reshape(EP // 128, 128)

  ids = x[:, 0:2].astype(jnp.int32)
  ids0 = jnp.pad(ids[:, 0], (0, NP - N)).reshape(NP // 128, 128)
  ids1 = jnp.pad(ids[:, 1], (0, NP - N)).reshape(NP // 128, 128)
  xf = jnp.pad(x[:, 2:5], ((0, NP - N), (0, 0)))

  zeros16 = jnp.zeros((NP, 16), jnp.float32)
  zeros32 = jnp.zeros((NP, 32), jnp.float32)
  ones16 = jnp.ones((128, 16), jnp.float32)

  embp = jnp.pad(emb, ((0, 0), (0, 2)))
  w1a = jnp.pad(W1[0:62], ((0, 2), (0, 0)))
  w1b = jnp.pad(W1[62:124], ((0, 2), (0, 0)))
  w1c = W1[124:127]
  w5p = jnp.pad(W5, ((0, 0), (0, 15)))

  ef0, ef1, degp = _sc_prep(embp, ids0, ids1, dstr, zeros16, ones16)

  g1a, g1b, dinv = _tc1(ef0, ef1, xf, degp, w1a, w1b, w1c)
  s1a = _sc_spmm(g1a, srcr, dstr, zeros32)
  s1b = _sc_spmm(g1b, srcr, dstr, zeros32)

  g2 = _tc2(s1a, s1b, g1a, g1b, dinv, b1.reshape(1, 64), W2)
  s2 = _sc_spmm(g2, srcr, dstr, zeros32)

  g3 = _tc_mid(s2, g2, dinv, b2.reshape(1, 32), W3, residual=False, f_out=32)
  s3 = _sc_spmm(g3, srcr, dstr, zeros32)

  g4 = _tc_mid(s3, g3, dinv, b3.reshape(1, 32), W4, residual=True, f_out=32)
  s4 = _sc_spmm(g4, srcr, dstr, zeros32)

  g5 = _tc_mid(s4, g4, dinv, b4.reshape(1, 32), w5p, residual=True, f_out=16)
  s5 = _sc_spmm(g5, srcr, dstr, zeros16)

  out = _tc_final(s5, g5, dinv, b5.reshape(1, 1), fcW1,
                  fcb1.reshape(1, 128), fcW2, fcb2.reshape(1, 128))
  return out.reshape(128)


# speedup vs baseline: 1.0239x; 1.0239x over previous
import functools

import jax
import jax.numpy as jnp
from jax import lax
from jax.experimental import pallas as pl
from jax.experimental.pallas import tpu as pltpu
from jax.experimental.pallas import tpu_sc as plsc

N = 38333
NP = 38400
E = 1226656
C = 1024
CR = C // 128
NCH0 = 42
NCH1 = 33
NCHT = NCH0 + NCH1
EP = 16 * NCHT * C
RPS = NP // 16
BT = 2400
GRID = NP // BT


def _mesh():
  return plsc.VectorSubcoreMesh(core_axis_name="c", subcore_axis_name="s")


def _sc_prep(emb, ids0, ids1, dstr, zeros1, ones1):
  ed = emb.shape[1]
  idrows = NP // 128

  def body(emb_h, ids0_h, ids1_h, dst_h, z_h, one_h, ef0_h, ef1_h, deg_h,
           idx_v, er0_v, er1_v, dst_v, ones_v, dacc, sems, ssems, isems):
    cid = lax.axis_index("c")
    sid = lax.axis_index("s")
    pltpu.sync_copy(z_h.at[pl.ds(sid * RPS, RPS)],
                    dacc.at[pl.ds(sid * RPS, RPS)])
    pltpu.sync_copy(one_h, ones_v)

    nr = lax.select(cid == 0, 14, 5)
    rbase = cid * (16 * 14) + sid * nr

    def erow(t, carry):
      row = rbase + t

      @pl.when(row < idrows)
      def _():
        @pl.when(t > 0)
        def _():
          rowp = row - 1
          pltpu.make_async_copy(er0_v, ef0_h.at[pl.ds(rowp * 128, 128)],
                                sems.at[2]).wait()
          pltpu.make_async_copy(er1_v, ef1_h.at[pl.ds(rowp * 128, 128)],
                                sems.at[3]).wait()
        pltpu.sync_copy(ids0_h.at[pl.ds(row, 1)], idx_v.at[pl.ds(0, 1)])
        pltpu.sync_copy(ids1_h.at[pl.ds(row, 1)], idx_v.at[pl.ds(1, 1)])
        g0 = pltpu.async_copy(emb_h.at[idx_v.at[0]], er0_v, sems.at[0])
        g1 = pltpu.async_copy(emb_h.at[idx_v.at[1]], er1_v, sems.at[1])
        g0.wait()
        pltpu.async_copy(er0_v, ef0_h.at[pl.ds(row * 128, 128)], sems.at[2])
        g1.wait()
        pltpu.async_copy(er1_v, ef1_h.at[pl.ds(row * 128, 128)], sems.at[3])
      return carry

    lax.fori_loop(0, nr, erow, 0)

    @pl.when(rbase < idrows)
    def _():
      rowl = lax.min(rbase + nr, idrows) - 1
      pltpu.make_async_copy(er0_v, ef0_h.at[pl.ds(rowl * 128, 128)],
                            sems.at[2]).wait()
      pltpu.make_async_copy(er1_v, ef1_h.at[pl.ds(rowl * 128, 128)],
                            sems.at[3]).wait()
    plsc.subcore_barrier()

    nc = lax.select(cid == 0, NCH0, NCH1)
    ebase = cid * (16 * NCH0 * CR) + sid * nc * CR
    pltpu.async_copy(dst_h.at[pl.ds(ebase, CR)], dst_v.at[0], isems.at[0])

    def chunk(k, carry):
      p = lax.rem(k, 2)
      pn = lax.rem(k + 1, 2)
      row0 = ebase + k * CR
      pltpu.make_async_copy(dst_h.at[pl.ds(row0, CR)], dst_v.at[p],
                            isems.at[p]).wait()
      for j in range(CR):
        @pl.when(k > 0)
        def _(j=j, p=p):
          pltpu.make_async_copy(ones_v, dacc.at[dst_v.at[p, j]],
                                ssems.at[j]).wait()
        pltpu.async_copy(ones_v, dacc.at[dst_v.at[p, j]], ssems.at[j],
                         add=True)

      @pl.when(k + 1 < nc)
      def _():
        rown = ebase + (k + 1) * CR
        pltpu.async_copy(dst_h.at[pl.ds(rown, CR)], dst_v.at[pn],
                         isems.at[pn])
      return carry

    lax.fori_loop(0, nc, chunk, 0)
    pl_ = lax.rem(nc - 1, 2)
    for j in range(CR):
      pltpu.make_async_copy(ones_v, dacc.at[dst_v.at[pl_, j]],
                            ssems.at[j]).wait()
    plsc.subcore_barrier()
    pltpu.sync_copy(dacc.at[pl.ds(sid * RPS, RPS)],
                    deg_h.at[cid, pl.ds(sid * RPS, RPS)])

  k = pl.kernel(
      body,
      out_type=(jax.ShapeDtypeStruct((NP, ed), jnp.float32),
                jax.ShapeDtypeStruct((NP, ed), jnp.float32),
                jax.ShapeDtypeStruct((2, NP, 16), jnp.float32)),
      mesh=_mesh(),
      compiler_params=pltpu.CompilerParams(use_tc_tiling_on_sc=False),
      scratch_types=[pltpu.VMEM((2, 128), jnp.int32),
                     pltpu.VMEM((128, ed), jnp.float32),
                     pltpu.VMEM((128, ed), jnp.float32),
                     pltpu.VMEM((2, CR, 128), jnp.int32),
                     pltpu.VMEM((128, 16), jnp.float32),
                     pltpu.VMEM_SHARED((NP, 16), jnp.float32),
                     pltpu.SemaphoreType.DMA((4,)),
                     pltpu.SemaphoreType.DMA((CR,)),
                     pltpu.SemaphoreType.DMA((2,))])
  return k(emb, ids0, ids1, dstr, zeros1, ones1)


def _sc_spmm(g, srcr, dstr, zeros):
  f_dim = g.shape[1]

  def body(g_h, src_h, dst_h, z_h, out_h, src_v, dst_v, rows_v, acc,
           gsems, ssems, isems):
    cid = lax.axis_index("c")
    sid = lax.axis_index("s")
    nc = lax.select(cid == 0, NCH0, NCH1)
    base = cid * (16 * NCH0 * CR) + sid * nc * CR

    pltpu.async_copy(src_h.at[pl.ds(base, CR)], src_v.at[0], isems.at[0])
    pltpu.async_copy(dst_h.at[pl.ds(base, CR)], dst_v.at[0], isems.at[1])
    pltpu.sync_copy(z_h.at[pl.ds(sid * RPS, RPS)],
                    acc.at[pl.ds(sid * RPS, RPS)])
    plsc.subcore_barrier()

    def chunk(k, carry):
      p = lax.rem(k, 2)
      pn = lax.rem(k + 1, 2)
      row0 = base + k * CR
      pltpu.make_async_copy(src_h.at[pl.ds(row0, CR)], src_v.at[p],
                            isems.at[2 * p]).wait()
      pltpu.make_async_copy(dst_h.at[pl.ds(row0, CR)], dst_v.at[p],
                            isems.at[2 * p + 1]).wait()
      gd = []
      for j in range(CR):
        @pl.when(k > 0)
        def _(j=j, p=p):
          pltpu.make_async_copy(rows_v.at[pl.ds(j * 128, 128)],
                                acc.at[dst_v.at[p, j]], ssems.at[j]).wait()
        gd.append(pltpu.async_copy(g_h.at[src_v.at[p, j]],
                                   rows_v.at[pl.ds(j * 128, 128)],
                                   gsems.at[j]))
      @pl.when(k + 1 < nc)
      def _():
        rown = base + (k + 1) * CR
        pltpu.async_copy(src_h.at[pl.ds(rown, CR)], src_v.at[pn],
                         isems.at[2 * pn])
        pltpu.async_copy(dst_h.at[pl.ds(rown, CR)], dst_v.at[pn],
                         isems.at[2 * pn + 1])
      for j in range(CR):
        gd[j].wait()
        pltpu.async_copy(rows_v.at[pl.ds(j * 128, 128)],
                         acc.at[dst_v.at[p, j]], ssems.at[j], add=True)
      return carry

    lax.fori_loop(0, nc, chunk, 0)
    pl_ = lax.rem(nc - 1, 2)
    for j in range(CR):
      pltpu.make_async_copy(rows_v.at[pl.ds(j * 128, 128)],
                            acc.at[dst_v.at[pl_, j]], ssems.at[j]).wait()
    plsc.subcore_barrier()
    pltpu.sync_copy(acc.at[pl.ds(sid * RPS, RPS)],
                    out_h.at[cid, pl.ds(sid * RPS, RPS)])

  k = pl.kernel(
      body,
      out_type=jax.ShapeDtypeStruct((2, NP, f_dim), jnp.float32),
      mesh=_mesh(),
      compiler_params=pltpu.CompilerParams(use_tc_tiling_on_sc=False),
      scratch_types=[pltpu.VMEM((2, CR, 128), jnp.int32),
                     pltpu.VMEM((2, CR, 128), jnp.int32),
                     pltpu.VMEM((C, f_dim), jnp.float32),
                     pltpu.VMEM_SHARED((NP, f_dim), jnp.float32),
                     pltpu.SemaphoreType.DMA((CR,)),
                     pltpu.SemaphoreType.DMA((CR,)),
                     pltpu.SemaphoreType.DMA((4,))])
  return k(g, srcr, dstr, zeros)


def _lrelu(v):
  return jnp.where(v > 0, v, 0.01 * v)


def _rowmask(i):
  rows = i * BT + lax.broadcasted_iota(jnp.int32, (BT, 1), 0)
  return rows < N


def _tc1(ef0, ef1, xf, degp, w1a, w1b, w1c):
  def body(ef0_r, ef1_r, xf_r, deg_r, wa_r, wb_r, wc_r, g1a_r, g1b_r, dinv_r):
    i = pl.program_id(0)
    deg = deg_r[0][:, 0:1] + deg_r[1][:, 0:1] + 1.0
    dinv = 1.0 / jnp.sqrt(deg)
    h = (jnp.dot(ef0_r[...], wa_r[...], preferred_element_type=jnp.float32)
         + jnp.dot(ef1_r[...], wb_r[...], preferred_element_type=jnp.float32)
         + jnp.dot(xf_r[...], wc_r[...], preferred_element_type=jnp.float32))
    g = jnp.where(_rowmask(i), dinv * h, 0.0)
    g1a_r[...] = g[:, :32]
    g1b_r[...] = g[:, 32:]
    dinv_r[...] = dinv

  ed = ef0.shape[1]
  return pl.pallas_call(
      body,
      grid=(GRID,),
      in_specs=[
          pl.BlockSpec((BT, ed), lambda i: (i, 0)),
          pl.BlockSpec((BT, ed), lambda i: (i, 0)),
          pl.BlockSpec((BT, 3), lambda i: (i, 0)),
          pl.BlockSpec((2, BT, 16), lambda i: (0, i, 0)),
          pl.BlockSpec((ed, 64), lambda i: (0, 0)),
          pl.BlockSpec((ed, 64), lambda i: (0, 0)),
          pl.BlockSpec((3, 64), lambda i: (0, 0)),
      ],
      out_specs=[
          pl.BlockSpec((BT, 32), lambda i: (i, 0)),
          pl.BlockSpec((BT, 32), lambda i: (i, 0)),
          pl.BlockSpec((BT, 1), lambda i: (i, 0)),
      ],
      out_shape=[
          jax.ShapeDtypeStruct((NP, 32), jnp.float32),
          jax.ShapeDtypeStruct((NP, 32), jnp.float32),
          jax.ShapeDtypeStruct((NP, 1), jnp.float32),
      ],
  )(ef0, ef1, xf, degp, w1a, w1b, w1c)


def _tc2(s1a, s1b, g1a, g1b, dinv, b1, w2):
  def body(sa_r, sb_r, ga_r, gb_r, dinv_r, b_r, w_r, out_r):
    i = pl.program_id(0)
    s = jnp.concatenate([sa_r[0] + sa_r[1] + ga_r[...],
                         sb_r[0] + sb_r[1] + gb_r[...]], axis=1)
    t = dinv_r[...] * s + b_r[...]
    f = _lrelu(t)
    out_r[...] = jnp.where(
        _rowmask(i),
        dinv_r[...] * jnp.dot(f, w_r[...], preferred_element_type=jnp.float32),
        0.0)

  return pl.pallas_call(
      body,
      grid=(GRID,),
      in_specs=[
          pl.BlockSpec((2, BT, 32), lambda i: (0, i, 0)),
          pl.BlockSpec((2, BT, 32), lambda i: (0, i, 0)),
          pl.BlockSpec((BT, 32), lambda i: (i, 0)),
          pl.BlockSpec((BT, 32), lambda i: (i, 0)),
          pl.BlockSpec((BT, 1), lambda i: (i, 0)),
          pl.BlockSpec((1, 64), lambda i: (0, 0)),
          pl.BlockSpec((64, 32), lambda i: (0, 0)),
      ],
      out_specs=pl.BlockSpec((BT, 32), lambda i: (i, 0)),
      out_shape=jax.ShapeDtypeStruct((NP, 32), jnp.float32),
  )(s1a, s1b, g1a, g1b, dinv, b1, w2)


def _tc_mid(s, g, dinv, b, w, residual, f_out):
  f_in = g.shape[1]

  def body(s_r, g_r, dinv_r, b_r, w_r, out_r):
    i = pl.program_id(0)
    t = dinv_r[...] * (s_r[0] + s_r[1] + g_r[...]) + b_r[...]
    f = _lrelu(t) + t if residual else _lrelu(t)
    out_r[...] = jnp.where(
        _rowmask(i),
        dinv_r[...] * jnp.dot(f, w_r[...], preferred_element_type=jnp.float32),
        0.0)

  return pl.pallas_call(
      body,
      grid=(GRID,),
      in_specs=[
          pl.BlockSpec((2, BT, f_in), lambda i: (0, i, 0)),
          pl.BlockSpec((BT, f_in), lambda i: (i, 0)),
          pl.BlockSpec((BT, 1), lambda i: (i, 0)),
          pl.BlockSpec((1, f_in), lambda i: (0, 0)),
          pl.BlockSpec((f_in, f_out), lambda i: (0, 0)),
      ],
      out_specs=pl.BlockSpec((BT, f_out), lambda i: (i, 0)),
      out_shape=jax.ShapeDtypeStruct((NP, f_out), jnp.float32),
  )(s, g, dinv, b, w)


def _tc_final(s5, g5, dinv, b5, fcw1, fcb1, fcw2, fcb2):
  def body(s_r, g_r, dinv_r, b5_r, w1_r, fb1_r, w2_r, fb2_r, out_r, acc):
    i = pl.program_id(0)

    @pl.when(i == 0)
    def _():
      acc[...] = jnp.zeros_like(acc)

    mask = _rowmask(i)
    t = (dinv_r[...] * (s_r[0][:, 0:1] + s_r[1][:, 0:1] + g_r[:, 0:1])
         + b5_r[...])
    f = jnp.where(mask, _lrelu(t), 0.0)
    w = jnp.where(mask, w1_r[...], 0.0)
    acc[...] += lax.dot_general(f, w, (((0,), (0,)), ((), ())),
                                preferred_element_type=jnp.float32)

    @pl.when(i == GRID - 1)
    def _():
      h = jnp.maximum(acc[...] + fb1_r[...], 0.0)
      out_r[...] = jnp.maximum(
          jnp.dot(h, w2_r[...], preferred_element_type=jnp.float32)
          + fb2_r[...], 0.0)

  return pl.pallas_call(
      body,
      grid=(GRID,),
      in_specs=[
          pl.BlockSpec((2, BT, 16), lambda i: (0, i, 0)),
          pl.BlockSpec((BT, 16), lambda i: (i, 0)),
          pl.BlockSpec((BT, 1), lambda i: (i, 0)),
          pl.BlockSpec((1, 1), lambda i: (0, 0)),
          pl.BlockSpec((BT, 128), lambda i: (i, 0)),
          pl.BlockSpec((1, 128), lambda i: (0, 0)),
          pl.BlockSpec((128, 128), lambda i: (0, 0)),
          pl.BlockSpec((1, 128), lambda i: (0, 0)),
      ],
      out_specs=pl.BlockSpec((1, 128), lambda i: (0, 0)),
      out_shape=jax.ShapeDtypeStruct((1, 128), jnp.float32),
      scratch_shapes=[pltpu.VMEM((1, 128), jnp.float32)],
  )(s5, g5, dinv, b5, fcw1, fcb1, fcw2, fcb2)


def kernel(x, edge_index, emb, W1, b1, W2, b2, W3, b3, W4, b4, W5, b5,
           fcW1, fcb1, fcW2, fcb2):
  src = edge_index[0].astype(jnp.int32)
  dst = edge_index[1].astype(jnp.int32)
  pad_e = jnp.full((EP - E,), N, jnp.int32)
  srcr = jnp.concatenate([src, pad_e]).reshape(EP // 128, 128)
  dstr = jnp.concatenate([dst, pad_e]).reshape(EP // 128, 128)

  ids = x[:, 0:2].astype(jnp.int32)
  ids0 = jnp.pad(ids[:, 0], (0, NP - N)).reshape(NP // 128, 128)
  ids1 = jnp.pad(ids[:, 1], (0, NP - N)).reshape(NP // 128, 128)
  xf = jnp.pad(x[:, 2:5], ((0, NP - N), (0, 0)))

  zeros16 = jnp.zeros((NP, 16), jnp.float32)
  zeros32 = jnp.zeros((NP, 32), jnp.float32)
  ones16 = jnp.ones((128, 16), jnp.float32)

  embp = jnp.pad(emb, ((0, 0), (0, 2)))
  w1a = jnp.pad(W1[0:62], ((0, 2), (0, 0)))
  w1b = jnp.pad(W1[62:124], ((0, 2), (0, 0)))
  w1c = W1[124:127]
  w5p = jnp.pad(W5, ((0, 0), (0, 15)))

  ef0, ef1, degp = _sc_prep(embp, ids0, ids1, dstr, zeros16, ones16)

  g1a, g1b, dinv = _tc1(ef0, ef1, xf, degp, w1a, w1b, w1c)
  s1a = _sc_spmm(g1a, srcr, dstr, zeros32)
  s1b = _sc_spmm(g1b, srcr, dstr, zeros32)

  g2 = _tc2(s1a, s1b, g1a, g1b, dinv, b1.reshape(1, 64), W2)
  s2 = _sc_spmm(g2, srcr, dstr, zeros32)

  g3 = _tc_mid(s2, g2, dinv, b2.reshape(1, 32), W3, residual=False, f_out=32)
  s3 = _sc_spmm(g3, srcr, dstr, zeros32)

  g4 = _tc_mid(s3, g3, dinv, b3.reshape(1, 32), W4, residual=True, f_out=32)
  s4 = _sc_spmm(g4, srcr, dstr, zeros32)

  g5 = _tc_mid(s4, g4, dinv, b4.reshape(1, 32), w5p, residual=True, f_out=16)
  s5 = _sc_spmm(g5, srcr, dstr, zeros16)

  out = _tc_final(s5, g5, dinv, b5.reshape(1, 1), fcW1,
                  fcb1.reshape(1, 128), fcW2, fcb2.reshape(1, 128))
  return out.reshape(128)
